# Initial kernel scaffold; baseline (speedup 1.0000x reference)
import jax
import jax.numpy as jnp
from jax.experimental import pallas as pl


def kernel(waveforms):
    raise NotImplementedError("write your pallas kernel here")


# speedup vs baseline: 4.4963x; 4.4963x over previous
import functools

import jax
import jax.numpy as jnp
from jax import lax
from jax.experimental import pallas as pl
from jax.experimental.pallas import tpu as pltpu
from jax.experimental.pallas import tpu_sc as plsc

_JITTER = 0.1
_SAMPLE_RATE = 2048.0
_PAD = int(_JITTER * _SAMPLE_RATE)

_NC, _NS = 2, 16
_NW = _NC * _NS

_B, _C, _T = 512, 2, 8192
_R = _B * _C
_RPW = _R // _NW
_PADB = 256
_BUF = _T + 2 * _PADB
_EDGE = 208

_mesh = plsc.VectorSubcoreMesh(
    core_axis_name="c", subcore_axis_name="s",
    num_cores=_NC, num_subcores=_NS,
)


@functools.partial(
    pl.kernel,
    out_type=jax.ShapeDtypeStruct((_R, _T), jnp.float32),
    mesh=_mesh,
    scratch_types=[
        pltpu.VMEM((_BUF,), jnp.float32),
        pltpu.VMEM((_T,), jnp.float32),
        pltpu.VMEM((_RPW,), jnp.int32),
    ],
    compiler_params=pltpu.CompilerParams(needs_layout_passes=False),
)
def _shift_rows(rows_hbm, src0_hbm, out_hbm, buf_v, out_v, src0_v):
    wid = lax.axis_index("s") * _NC + lax.axis_index("c")
    base = wid * _RPW
    pltpu.sync_copy(src0_hbm.at[pl.ds(base, _RPW)], src0_v)
    zeros = jnp.zeros((16,), jnp.float32)
    for j in range(_EDGE // 16):
        buf_v[pl.ds(_PADB - _EDGE + j * 16, 16)] = zeros
        buf_v[pl.ds(_PADB + _T + j * 16, 16)] = zeros
    iota = lax.iota(jnp.int32, 16)
    for i in range(_RPW):
        s0 = src0_v[pl.ds((i // 16) * 16, 16)][i % 16]
        pltpu.sync_copy(rows_hbm.at[base + i], buf_v.at[pl.ds(_PADB, _T)])
        row_vec = iota + s0

        @plsc.parallel_loop(0, _T, step=16, unroll=8)
        def chunk(t0, _rv=row_vec):
            out_v[pl.ds(t0, 16)] = plsc.load_gather(buf_v, [_rv + t0])

        pltpu.sync_copy(out_v, out_hbm.at[base + i])


def kernel(waveforms):
    B, C, T = waveforms.shape
    rkey = jax.random.key(42)
    shifts = jax.random.uniform(rkey, (B,), dtype=jnp.float32)
    shifts = 2.0 * _JITTER * shifts - _JITTER
    shifts = shifts * _SAMPLE_RATE
    shifts = shifts.astype(jnp.int32)
    src0 = jnp.repeat(_PADB + shifts, C)
    rows = waveforms.reshape(B * C, T)
    out = _shift_rows(rows, src0)
    return out.reshape(B, C, T)


# speedup vs baseline: 5.4277x; 1.2072x over previous
import functools

import jax
import jax.numpy as jnp
from jax import lax
from jax.experimental import pallas as pl
from jax.experimental.pallas import tpu as pltpu
from jax.experimental.pallas import tpu_sc as plsc

_JITTER = 0.1
_SAMPLE_RATE = 2048.0
_PAD = int(_JITTER * _SAMPLE_RATE)

_NC, _NS = 2, 16
_NW = _NC * _NS

_B, _C, _T = 512, 2, 8192
_R = _B * _C
_RPW = _R // _NW
_EDGE = 208
_PADB = 256
_BUF = _T + 2 * _PADB

_mesh = plsc.VectorSubcoreMesh(
    core_axis_name="c", subcore_axis_name="s",
    num_cores=_NC, num_subcores=_NS,
)


@functools.partial(
    pl.kernel,
    out_type=jax.ShapeDtypeStruct((_R, _T), jnp.float32),
    mesh=_mesh,
    scratch_types=[
        pltpu.VMEM((_BUF,), jnp.float32),
        pltpu.VMEM((_BUF,), jnp.float32),
        pltpu.VMEM((_T,), jnp.float32),
        pltpu.VMEM((_T,), jnp.float32),
        pltpu.VMEM((_RPW,), jnp.int32),
        pltpu.SemaphoreType.DMA,
        pltpu.SemaphoreType.DMA,
        pltpu.SemaphoreType.DMA,
        pltpu.SemaphoreType.DMA,
    ],
    compiler_params=pltpu.CompilerParams(needs_layout_passes=False),
)
def _shift_rows(rows_hbm, src0_hbm, out_hbm, in_v0, in_v1, out_v0, out_v1,
                src0_v, sin0, sin1, sout0, sout1):
    in_v = (in_v0, in_v1)
    out_v = (out_v0, out_v1)
    sin = (sin0, sin1)
    sout = (sout0, sout1)
    wid = lax.axis_index("s") * _NC + lax.axis_index("c")
    base = wid * _RPW
    pltpu.sync_copy(src0_hbm.at[pl.ds(base, _RPW)], src0_v)
    zeros = jnp.zeros((16,), jnp.float32)
    for b in range(2):
        for j in range(_EDGE // 16):
            in_v[b][pl.ds(_PADB - _EDGE + j * 16, 16)] = zeros
            in_v[b][pl.ds(_PADB + _T + j * 16, 16)] = zeros
    iota = lax.iota(jnp.int32, 16)

    def in_copy(i, b):
        return pltpu.make_async_copy(
            rows_hbm.at[base + i], in_v[b].at[pl.ds(_PADB, _T)], sin[b])

    def out_copy(i, b):
        return pltpu.make_async_copy(
            out_v[b], out_hbm.at[base + i], sout[b])

    in_copy(0, 0).start()
    in_copy(1, 1).start()
    for i in range(_RPW):
        b = i & 1
        in_copy(i, b).wait()
        if i >= 2:
            out_copy(i - 2, b).wait()
        s0 = src0_v[pl.ds((i // 16) * 16, 16)][i % 16]
        row_vec = iota + s0

        @plsc.parallel_loop(0, _T, step=16, unroll=8)
        def chunk(t0, _rv=row_vec, _b=b):
            out_v[_b][pl.ds(t0, 16)] = plsc.load_gather(
                in_v[_b], [_rv + t0])

        if i + 2 < _RPW:
            in_copy(i + 2, b).start()
        out_copy(i, b).start()
    out_copy(_RPW - 2, 0).wait()
    out_copy(_RPW - 1, 1).wait()


def kernel(waveforms):
    B, C, T = waveforms.shape
    rkey = jax.random.key(42)
    shifts = jax.random.uniform(rkey, (B,), dtype=jnp.float32)
    shifts = 2.0 * _JITTER * shifts - _JITTER
    shifts = shifts * _SAMPLE_RATE
    shifts = shifts.astype(jnp.int32)
    src0 = jnp.repeat(_PADB + shifts, C)
    rows = waveforms.reshape(B * C, T)
    out = _shift_rows(rows, src0)
    return out.reshape(B, C, T)


# speedup vs baseline: 13.6785x; 2.5201x over previous
import functools

import jax
import jax.numpy as jnp
from jax import lax
from jax.experimental import pallas as pl
from jax.experimental.pallas import tpu as pltpu
from jax.experimental.pallas import tpu_sc as plsc

_JITTER = 0.1
_SAMPLE_RATE = 2048.0
_PAD = int(_JITTER * _SAMPLE_RATE)

_NC, _NS = 2, 16
_NW = _NC * _NS

_B, _C, _T = 512, 2, 8192
_R = _B * _C
_RPW = _R // _NW
_BPW = _B // _NW
_EDGE = 208
_PADB = 256
_BUF = _T + 2 * _PADB

_mesh = plsc.VectorSubcoreMesh(
    core_axis_name="c", subcore_axis_name="s",
    num_cores=_NC, num_subcores=_NS,
)


@functools.partial(
    pl.kernel,
    out_type=jax.ShapeDtypeStruct((_B, _C, _T), jnp.float32),
    mesh=_mesh,
    scratch_types=[
        pltpu.VMEM((_BUF,), jnp.float32),
        pltpu.VMEM((_BUF,), jnp.float32),
        pltpu.VMEM((_T,), jnp.float32),
        pltpu.VMEM((_T,), jnp.float32),
        pltpu.VMEM((16,), jnp.int32),
        pltpu.SemaphoreType.DMA,
        pltpu.SemaphoreType.DMA,
        pltpu.SemaphoreType.DMA,
        pltpu.SemaphoreType.DMA,
    ],
    compiler_params=pltpu.CompilerParams(needs_layout_passes=False),
)
def _shift_rows(w_hbm, src0_hbm, out_hbm, in_v0, in_v1, out_v0, out_v1,
                src0_v, sin0, sin1, sout0, sout1):
    in_v = (in_v0, in_v1)
    out_v = (out_v0, out_v1)
    sin = (sin0, sin1)
    sout = (sout0, sout1)
    wid = lax.axis_index("s") * _NC + lax.axis_index("c")
    base_b = wid * _BPW
    pltpu.sync_copy(src0_hbm.at[pl.ds(base_b, 16)], src0_v)
    zeros = jnp.zeros((16,), jnp.float32)
    for b in range(2):
        for j in range(_EDGE // 16):
            in_v[b][pl.ds(_PADB - _EDGE + j * 16, 16)] = zeros
            in_v[b][pl.ds(_PADB + _T + j * 16, 16)] = zeros
    iota = lax.iota(jnp.int32, 16)

    def in_copy(i, b):
        return pltpu.make_async_copy(
            w_hbm.at[base_b + i // _C, i % _C],
            in_v[b].at[pl.ds(_PADB, _T)], sin[b])

    def out_copy(i, b):
        return pltpu.make_async_copy(
            out_v[b], out_hbm.at[base_b + i // _C, i % _C], sout[b])

    in_copy(0, 0).start()
    in_copy(1, 1).start()
    for i in range(_RPW):
        b = i & 1
        in_copy(i, b).wait()
        if i >= 2:
            out_copy(i - 2, b).wait()
        s0 = src0_v[pl.ds(0, 16)][i // _C]
        row_vec = iota + s0

        @plsc.parallel_loop(0, _T, step=16, unroll=8)
        def chunk(t0, _rv=row_vec, _b=b):
            out_v[_b][pl.ds(t0, 16)] = plsc.load_gather(
                in_v[_b], [_rv + t0])

        if i + 2 < _RPW:
            in_copy(i + 2, b).start()
        out_copy(i, b).start()
    out_copy(_RPW - 2, 0).wait()
    out_copy(_RPW - 1, 1).wait()


def kernel(waveforms):
    B, C, T = waveforms.shape
    rkey = jax.random.key(42)
    shifts = jax.random.uniform(rkey, (B,), dtype=jnp.float32)
    shifts = 2.0 * _JITTER * shifts - _JITTER
    shifts = shifts * _SAMPLE_RATE
    shifts = shifts.astype(jnp.int32)
    src0 = _PADB + shifts
    return _shift_rows(waveforms, src0)


# speedup vs baseline: 14.7777x; 1.0804x over previous
import functools

import jax
import jax.numpy as jnp
from jax import lax
from jax.experimental import pallas as pl
from jax.experimental.pallas import tpu as pltpu
from jax.experimental.pallas import tpu_sc as plsc

_JITTER = 0.1
_SAMPLE_RATE = 2048.0
_PAD = int(_JITTER * _SAMPLE_RATE)

_NC, _NS = 2, 16
_NW = _NC * _NS

_B, _C, _T = 512, 2, 8192
_R = _B * _C
_RPW = _R // _NW
_EDGE = 208
_PADB = 256
_BUF = _T + 2 * _PADB

_mesh = plsc.VectorSubcoreMesh(
    core_axis_name="c", subcore_axis_name="s",
    num_cores=_NC, num_subcores=_NS,
)


@functools.partial(
    pl.kernel,
    out_type=jax.ShapeDtypeStruct((_B, _C, _T), jnp.float32),
    mesh=_mesh,
    scratch_types=[
        pltpu.VMEM((_BUF,), jnp.float32),
        pltpu.VMEM((_BUF,), jnp.float32),
        pltpu.VMEM((_T,), jnp.float32),
        pltpu.VMEM((_T,), jnp.float32),
        pltpu.VMEM((_RPW * 16,), jnp.int32),
        pltpu.SemaphoreType.DMA,
        pltpu.SemaphoreType.DMA,
        pltpu.SemaphoreType.DMA,
        pltpu.SemaphoreType.DMA,
    ],
    compiler_params=pltpu.CompilerParams(needs_layout_passes=False),
)
def _shift_rows(w_hbm, rv_hbm, out_hbm, in_v0, in_v1, out_v0, out_v1,
                rv_v, sin0, sin1, sout0, sout1):
    in_v = (in_v0, in_v1)
    out_v = (out_v0, out_v1)
    sin = (sin0, sin1)
    sout = (sout0, sout1)
    wid = lax.axis_index("s") * _NC + lax.axis_index("c")
    base = wid * _RPW
    pltpu.sync_copy(rv_hbm.at[pl.ds(base * 16, _RPW * 16)], rv_v)
    zeros = jnp.zeros((16,), jnp.float32)
    for b in range(2):
        for j in range(_EDGE // 16):
            in_v[b][pl.ds(_PADB - _EDGE + j * 16, 16)] = zeros
            in_v[b][pl.ds(_PADB + _T + j * 16, 16)] = zeros

    def in_copy(i, b):
        return pltpu.make_async_copy(
            w_hbm.at[(base + i) // _C, (base + i) % _C],
            in_v[b].at[pl.ds(_PADB, _T)], sin[b])

    def out_copy(i, b):
        return pltpu.make_async_copy(
            out_v[b], out_hbm.at[(base + i) // _C, (base + i) % _C], sout[b])

    in_copy(0, 0).start()
    in_copy(1, 1).start()

    def pair(k):
        for b in range(2):
            i = 2 * k + b
            in_copy(i, b).wait()

            @pl.when(k >= 1)
            def _():
                out_copy(i - 2, b).wait()

            rv = rv_v[pl.ds(pl.multiple_of(i * 16, 16), 16)]

            @plsc.parallel_loop(0, _T, step=16, unroll=8)
            def chunk(t0, _rv=rv, _b=b):
                out_v[_b][pl.ds(t0, 16)] = plsc.load_gather(
                    in_v[_b], [_rv + t0])

            @pl.when(i + 2 < _RPW)
            def _():
                in_copy(i + 2, b).start()

            out_copy(i, b).start()

    lax.fori_loop(0, _RPW // 2, lambda k, c: (pair(k), c)[1], 0)
    out_copy(_RPW - 2, 0).wait()
    out_copy(_RPW - 1, 1).wait()


def kernel(waveforms):
    B, C, T = waveforms.shape
    rkey = jax.random.key(42)
    shifts = jax.random.uniform(rkey, (B,), dtype=jnp.float32)
    shifts = 2.0 * _JITTER * shifts - _JITTER
    shifts = shifts * _SAMPLE_RATE
    shifts = shifts.astype(jnp.int32)
    src0 = jnp.repeat(_PADB + shifts, C)
    rv = src0[:, None] + jnp.arange(16, dtype=jnp.int32)[None, :]
    return _shift_rows(waveforms, rv.reshape(-1))
